# baseline (speedup 1.0000x reference)
import functools

import jax
import jax.numpy as jnp
from jax import lax
from jax.experimental import pallas as pl
from jax.experimental.pallas import tpu as pltpu
from jax.experimental.pallas import tpu_sc as plsc

B, L, NC = 4096, 50, 26
V = 100000
ED = 32
ND = 13
DM = 128

TOK = B * L
SLOTS = 32
IDX2 = TOK * SLOTS
LINES = IDX2 * ED // 128
IDX_PER_STREAM = 128
FIRES = 8
CHUNK = FIRES * IDX_PER_STREAM


def _sc_gather(flat_table, idx2d):
    info = plsc.get_sparse_core_info()
    nw = info.num_cores * info.num_subcores
    total_chunks = IDX2 // CHUNK
    assert total_chunks % nw == 0
    chunks_per_worker = total_chunks // nw

    mesh = plsc.VectorSubcoreMesh(core_axis_name="c", subcore_axis_name="s")

    @functools.partial(
        pl.kernel,
        out_type=jax.ShapeDtypeStruct((LINES, 128), jnp.float32),
        mesh=mesh,
        scratch_types=[
            pltpu.VMEM((FIRES, IDX_PER_STREAM), jnp.int32),
            pltpu.VMEM((CHUNK, ED), jnp.float32),
            pltpu.VMEM((CHUNK * ED // 128, 128), jnp.float32),
            pltpu.SemaphoreType.DMA,
        ],
        compiler_params=pltpu.CompilerParams(use_tc_tiling_on_sc=False),
    )
    def gather_kernel(table_hbm, idx_hbm, out_hbm, idx_v, rows_v, lines_v, sem):
        wid = lax.axis_index("s") * info.num_cores + lax.axis_index("c")
        lines_per_chunk = CHUNK * ED // 128

        def chunk_body(k, carry):
            cid = wid * chunks_per_worker + k
            idx_off = pl.multiple_of(cid * FIRES, 8)
            line_off = pl.multiple_of(cid * lines_per_chunk, 8)
            pltpu.sync_copy(idx_hbm.at[pl.ds(idx_off, FIRES)], idx_v)
            cps = []
            for j in range(FIRES):
                cps.append(pltpu.async_copy(
                    table_hbm.at[idx_v.at[j]],
                    rows_v.at[pl.ds(j * IDX_PER_STREAM, IDX_PER_STREAM)],
                    sem,
                ))
            for cp in cps:
                cp.wait()

            def repack_body(li, c2):
                for a in range(4):
                    for h in range(2):
                        lines_v[li, pl.ds(a * 32 + h * 16, 16)] = (
                            rows_v[li * 4 + a, pl.ds(h * 16, 16)])
                return c2

            lax.fori_loop(0, lines_per_chunk, repack_body, 0, unroll=4)
            pltpu.sync_copy(lines_v,
                            out_hbm.at[pl.ds(line_off, lines_per_chunk)])
            return carry

        lax.fori_loop(0, chunks_per_worker, chunk_body, 0, unroll=False)

    return gather_kernel(flat_table, idx2d)


def _tc_matmul_body(x_ref, num_ref, wp_ref, wn_ref, wf2_ref, bn_ref, bf_ref,
                    out_ref):
    acc = jnp.dot(x_ref[:, 0, :], wp_ref[pl.ds(0, 128), :],
                  preferred_element_type=jnp.float32)
    for r in range(1, 8):
        acc += jnp.dot(x_ref[:, r, :], wp_ref[pl.ds(r * 128, 128), :],
                       preferred_element_type=jnp.float32)
    num_proj = (
        jnp.dot(num_ref[...], wn_ref[...], preferred_element_type=jnp.float32)
        + bn_ref[...]
    )
    acc += jnp.dot(num_proj, wf2_ref[...], preferred_element_type=jnp.float32)
    out_ref[...] = acc + bf_ref[...]


def _tc_matmul(x3, num_flat, w_pad, w_num, wf_num, b_num, b_final):
    bm = 1024
    grid = (TOK // bm,)
    return pl.pallas_call(
        _tc_matmul_body,
        grid=grid,
        in_specs=[
            pl.BlockSpec((bm, 8, 128), lambda i: (i, 0, 0)),
            pl.BlockSpec((bm, ND), lambda i: (i, 0)),
            pl.BlockSpec((8 * 128, DM), lambda i: (0, 0)),
            pl.BlockSpec((ND, ED), lambda i: (0, 0)),
            pl.BlockSpec((ED, DM), lambda i: (0, 0)),
            pl.BlockSpec((1, ED), lambda i: (0, 0)),
            pl.BlockSpec((1, DM), lambda i: (0, 0)),
        ],
        out_specs=pl.BlockSpec((bm, DM), lambda i: (i, 0)),
        out_shape=jax.ShapeDtypeStruct((TOK, DM), jnp.float32),
    )(x3, num_flat, w_pad, w_num, wf_num, b_num, b_final)


def kernel(cat_feats, num_feats, emb_tables, W_num, b_num, W_final, b_final):
    flat_table = emb_tables.reshape(NC * (V + 1), ED)
    offsets = (jnp.arange(NC, dtype=jnp.int32) * (V + 1))[None, None, :]
    idx_real = cat_feats.astype(jnp.int32) + offsets
    idx_pad = jnp.full((B, L, SLOTS - NC), V, dtype=jnp.int32)
    idx2d = jnp.concatenate([idx_real, idx_pad], axis=-1).reshape(
        IDX2 // IDX_PER_STREAM, IDX_PER_STREAM)

    x3 = _sc_gather(flat_table, idx2d).reshape(TOK, 8, 128)

    w_pad = jnp.concatenate(
        [W_final[: NC * ED], jnp.zeros((8 * 128 - NC * ED, DM), jnp.float32)])

    out = _tc_matmul(
        x3,
        num_feats.reshape(TOK, ND),
        w_pad,
        W_num,
        W_final[NC * ED:],
        b_num.reshape(1, ED),
        b_final.reshape(1, DM),
    )
    return out.reshape(B, L, DM)
